# baseline (device time: 24548 ns/iter reference)
import jax
import jax.numpy as jnp
from jax import lax
from jax.experimental import pallas as pl
from jax.experimental.pallas import tpu as pltpu

N_DEV = 4
EPS = 1e-5
NCH = 4


def kernel(x, t_emb, W_scale, W_shift):
    b, s, c = x.shape
    c_global = c * N_DEV
    ch = s // NCH

    def body(x_ref, t_ref, wsc_ref, wsh_ref, out_ref,
             xv, obuf, stats_ref, peer_ref,
             in_sems, out_sems, send_sems, recv_sems):
        my_i = lax.axis_index("i")

        in_cps = []
        for bi in range(b):
            for k in range(NCH):
                cp = pltpu.make_async_copy(
                    x_ref.at[bi, pl.ds(k * ch, ch), :],
                    xv.at[bi, pl.ds(k * ch, ch), :],
                    in_sems.at[bi * NCH + k],
                )
                cp.start()
                in_cps.append(cp)

        for bi in range(b):
            for k in range(NCH):
                in_cps[bi * NCH + k].wait()
                xb = xv[bi, k * ch:(k + 1) * ch, :]
                stats_ref[bi:bi + 1, k * ch:(k + 1) * ch] = (
                    jnp.sum(xb, axis=1)[None, :])
                stats_ref[b + bi:b + bi + 1, k * ch:(k + 1) * ch] = (
                    jnp.sum(xb * xb, axis=1)[None, :])

        rdmas = []
        for d in range(1, N_DEV):
            tgt = lax.rem(my_i + d, N_DEV)
            rdma = pltpu.make_async_remote_copy(
                src_ref=stats_ref,
                dst_ref=peer_ref.at[d - 1],
                send_sem=send_sems.at[d - 1],
                recv_sem=recv_sems.at[d - 1],
                device_id=(tgt,),
                device_id_type=pl.DeviceIdType.MESH,
            )
            rdma.start()
            rdmas.append(rdma)

        scale = jnp.dot(t_ref[:, :], wsc_ref[:, :],
                        preferred_element_type=jnp.float32)
        shift = jnp.dot(t_ref[:, :], wsh_ref[:, :],
                        preferred_element_type=jnp.float32)

        for rdma in rdmas:
            rdma.wait()

        total = stats_ref[:, :]
        for d in range(1, N_DEV):
            total = total + peer_ref[d - 1]

        inv_c = 1.0 / c_global
        mean = total[0:b, :] * inv_c
        ex2 = total[b:2 * b, :] * inv_c
        var = ex2 - mean * mean
        rstd = lax.rsqrt(var + EPS)

        out_cps = [None, None]
        for bi in range(b):
            g = (1.0 + scale[bi])[None, :]
            sh = shift[bi][None, :]
            for k in range(NCH):
                idx = bi * NCH + k
                slot = idx % 2
                if out_cps[slot] is not None:
                    out_cps[slot].wait()
                xb = xv[bi, k * ch:(k + 1) * ch, :]
                m = mean[bi][k * ch:(k + 1) * ch][:, None]
                r = rstd[bi][k * ch:(k + 1) * ch][:, None]
                obuf[slot] = ((xb - m) * r * g + sh).astype(jnp.bfloat16)
                cp = pltpu.make_async_copy(
                    obuf.at[slot],
                    out_ref.at[bi, pl.ds(k * ch, ch), :],
                    out_sems.at[slot],
                )
                cp.start()
                out_cps[slot] = cp
        for cp in out_cps:
            cp.wait()

    return pl.pallas_call(
        body,
        out_shape=jax.ShapeDtypeStruct((b, s, c), jnp.bfloat16),
        in_specs=[
            pl.BlockSpec(memory_space=pl.ANY),
            pl.BlockSpec(memory_space=pltpu.VMEM),
            pl.BlockSpec(memory_space=pltpu.VMEM),
            pl.BlockSpec(memory_space=pltpu.VMEM),
        ],
        out_specs=pl.BlockSpec(memory_space=pl.ANY),
        scratch_shapes=[
            pltpu.VMEM((b, s, c), jnp.float32),
            pltpu.VMEM((2, ch, c), jnp.bfloat16),
            pltpu.VMEM((2 * b, s), jnp.float32),
            pltpu.VMEM((N_DEV - 1, 2 * b, s), jnp.float32),
            pltpu.SemaphoreType.DMA((b * NCH,)),
            pltpu.SemaphoreType.DMA((2,)),
            pltpu.SemaphoreType.DMA((N_DEV - 1,)),
            pltpu.SemaphoreType.DMA((N_DEV - 1,)),
        ],
    )(x, t_emb, W_scale, W_shift)
